# baseline (speedup 1.0000x reference)
import functools

import jax
import jax.numpy as jnp
from jax import lax
from jax.experimental import pallas as pl
from jax.experimental.pallas import tpu as pltpu
from jax.experimental.pallas import tpu_sc as plsc

_B = 4096
_L = 200
_CTX = 32
_NW = 32
_UR = 128
_APW = _L * _B // _UR // _NW
_UPW = _APW + 1
_IDXPW = _APW * _UR


def _tanh16(x):
    e = jnp.exp(x + x)
    return 1.0 - 2.0 / (e + 1.0)


def _body(topics_hbm, absidx_hbm, table_hbm, out1_hbm, out2_hbm,
          idx_v, idx4_v, g_v, o_v, gsem0, gsem1, osem0, osem1):
    wid = lax.axis_index("s") * 2 + lax.axis_index("c")
    gsems = (gsem0, gsem1)
    osems = (osem0, osem1)

    pltpu.sync_copy(absidx_hbm.at[pl.ds(wid * _IDXPW, _IDXPW)],
                    idx_v.at[pl.ds(0, _IDXPW)])
    pltpu.sync_copy(topics_hbm.at[pl.ds(wid * _UR, _UR)],
                    idx_v.at[pl.ds(_IDXPW, _UR)])

    def prep_and_start_gather(k, b):
        for j in range(_UR // 16):
            v = idx_v[pl.ds(k * _UR + j * 16, 16)]
            idx4_v[b, pl.ds(j * 16, 16)] = v >> 2
        pltpu.async_copy(table_hbm.at[idx4_v.at[b]], g_v.at[b], gsems[b])

    def wait_gather(k, b):
        pltpu.make_async_copy(table_hbm.at[idx4_v.at[b]], g_v.at[b],
                              gsems[b]).wait()

    def write_out(k, b):
        @pl.when(k < _APW)
        def _():
            pltpu.async_copy(
                o_v.at[b],
                out2_hbm.at[pl.ds(wid * _IDXPW + k * _UR, _UR)], osems[b])

        @pl.when(k >= _APW)
        def _():
            pltpu.async_copy(
                o_v.at[b], out1_hbm.at[pl.ds(wid * _UR, _UR)], osems[b])

    def wait_out(k, b):
        @pl.when(k < _APW)
        def _():
            pltpu.make_async_copy(
                o_v.at[b],
                out2_hbm.at[pl.ds(wid * _IDXPW + k * _UR, _UR)],
                osems[b]).wait()

        @pl.when(k >= _APW)
        def _():
            pltpu.make_async_copy(
                o_v.at[b], out1_hbm.at[pl.ds(wid * _UR, _UR)],
                osems[b]).wait()

    def compute_unit(k, b):
        def row_iter(j, c):
            r0 = j * 16
            sv = idx_v[pl.ds(k * _UR + r0, 16)]
            bases = (sv & 3) << 5
            for t in range(16):
                r = r0 + t
                base = bases[t]
                x0 = g_v[b, r, pl.ds(base, 16)]
                x1 = g_v[b, r, pl.ds(base + 16, 16)]
                o_v[b, r, pl.ds(0, 16)] = _tanh16(x0)
                o_v[b, r, pl.ds(16, 16)] = _tanh16(x1)
            return c

        lax.fori_loop(0, _UR // 16, row_iter, 0)

    prep_and_start_gather(0, 0)

    def unit_pair_iter(k2, carry):
        for b in range(2):
            k = 2 * k2 + b
            nb = 1 - b

            @pl.when(k + 1 < _UPW)
            def _(k=k, nb=nb):
                prep_and_start_gather(k + 1, nb)

            wait_gather(k, b)

            @pl.when(k >= 2)
            def _(k=k, b=b):
                wait_out(k - 2, b)

            compute_unit(k, b)
            write_out(k, b)

        return carry

    lax.fori_loop(0, _UPW // 2, unit_pair_iter, 0)

    k_last = _UPW - 1
    wait_gather(k_last, 0)
    wait_out(k_last - 2, 0)
    compute_unit(k_last, 0)
    write_out(k_last, 0)

    wait_out(_UPW - 2, 1)
    wait_out(_UPW - 1, 0)


_mesh = plsc.VectorSubcoreMesh(core_axis_name="c", subcore_axis_name="s")

_gather_tanh = functools.partial(
    pl.kernel,
    out_type=(
        jax.ShapeDtypeStruct((_B, _CTX), jnp.float32),
        jax.ShapeDtypeStruct((_L * _B, _CTX), jnp.float32),
    ),
    mesh=_mesh,
    scratch_types=[
        pltpu.VMEM((_IDXPW + _UR,), jnp.int32),
        pltpu.VMEM((2, _UR), jnp.int32),
        pltpu.VMEM((2, _UR, 128), jnp.float32),
        pltpu.VMEM((2, _UR, _CTX), jnp.float32),
        pltpu.SemaphoreType.DMA,
        pltpu.SemaphoreType.DMA,
        pltpu.SemaphoreType.DMA,
        pltpu.SemaphoreType.DMA,
    ],
    compiler_params=pltpu.CompilerParams(
        use_tc_tiling_on_sc=True, needs_layout_passes=False),
)(_body)


def kernel(topics, structure_abstracts, embedding):
    absidx = structure_abstracts.T.reshape(-1).astype(jnp.int32)
    table = embedding.reshape(250000, 128)
    o1, o2 = _gather_tanh(topics.astype(jnp.int32), absidx, table)
    out1 = o1.reshape(_B, 1, _CTX)
    out2 = o2.reshape(_L, _B, _CTX).transpose(1, 0, 2)
    return (out1, out2)


# speedup vs baseline: 1.7116x; 1.7116x over previous
import functools

import jax
import jax.numpy as jnp
from jax import lax
from jax.experimental import pallas as pl
from jax.experimental.pallas import tpu as pltpu
from jax.experimental.pallas import tpu_sc as plsc

_B = 4096
_L = 200
_CTX = 32
_NW = 32
_UR = 1024
_NU2 = _L * _B // _UR
_NU = _NU2 + _B // _UR
_UPW = -(-_NU // _NW)


def _tanh16(x):
    e = jnp.exp(x + x)
    return 1.0 - 2.0 / (e + 1.0)


def _body(topics_hbm, absidx_hbm, table_hbm, out1_hbm, out2_hbm,
          idx_v, g_v, sem0, sem1, osem0, osem1):
    wid = lax.axis_index("s") * 2 + lax.axis_index("c")
    gsems = (sem0, sem1)
    osems = (osem0, osem1)

    def load_idx(k, buf):
        u = wid + _NW * k

        @pl.when(u < _NU2)
        def _():
            pltpu.sync_copy(absidx_hbm.at[pl.ds(u * _UR, _UR)],
                            idx_v.at[buf])

        @pl.when(jnp.logical_and(u >= _NU2, u < _NU))
        def _():
            pltpu.sync_copy(topics_hbm.at[pl.ds((u - _NU2) * _UR, _UR)],
                            idx_v.at[buf])

    def start_gather(k, buf):
        @pl.when(wid + _NW * k < _NU)
        def _():
            pltpu.async_copy(table_hbm.at[idx_v.at[buf]], g_v.at[buf],
                             gsems[buf])

    def wait_gather(buf):
        pltpu.make_async_copy(table_hbm.at[idx_v.at[buf]], g_v.at[buf],
                              gsems[buf]).wait()

    def write_out(k, buf):
        u = wid + _NW * k

        @pl.when(u < _NU2)
        def _():
            pltpu.async_copy(
                g_v.at[buf],
                out2_hbm.at[pl.ds(u * _UR, _UR), pl.ds(0, _CTX)],
                osems[buf])

        @pl.when(jnp.logical_and(u >= _NU2, u < _NU))
        def _():
            pltpu.async_copy(
                g_v.at[buf],
                out1_hbm.at[pl.ds((u - _NU2) * _UR, _UR), pl.ds(0, _CTX)],
                osems[buf])

    def wait_out(k, buf):
        u = wid + _NW * k

        @pl.when(u < _NU2)
        def _():
            pltpu.make_async_copy(
                g_v.at[buf],
                out2_hbm.at[pl.ds(u * _UR, _UR), pl.ds(0, _CTX)],
                osems[buf]).wait()

        @pl.when(jnp.logical_and(u >= _NU2, u < _NU))
        def _():
            pltpu.make_async_copy(
                g_v.at[buf],
                out1_hbm.at[pl.ds((u - _NU2) * _UR, _UR), pl.ds(0, _CTX)],
                osems[buf]).wait()

    load_idx(0, 0)
    start_gather(0, 0)

    def unit_pair_iter(k2, carry):
        for b in range(2):
            k = 2 * k2 + b
            nb = 1 - b
            u = wid + _NW * k

            @pl.when(k >= 1)
            def _(k=k, nb=nb):
                wait_out(k - 1, nb)

            @pl.when(u + _NW < _NU)
            def _(k=k, nb=nb):
                load_idx(k + 1, nb)

            start_gather(k + 1, nb)

            @pl.when(u < _NU)
            def _run_unit(k=k, b=b):
                wait_gather(b)

                def row_iter(j, c):
                    r0 = j * 4
                    for t in range(4):
                        for h in range(2):
                            sl = (b, r0 + t, pl.ds(16 * h, 16))
                            g_v[sl] = _tanh16(g_v[sl])
                    return c

                lax.fori_loop(0, _UR // 4, row_iter, 0)
                write_out(k, b)

        return carry

    lax.fori_loop(0, _UPW // 2, unit_pair_iter, 0)
    wait_out(_UPW - 1, (_UPW - 1) % 2)


_mesh = plsc.VectorSubcoreMesh(core_axis_name="c", subcore_axis_name="s")

_gather_tanh = functools.partial(
    pl.kernel,
    out_type=(
        jax.ShapeDtypeStruct((_B, 128), jnp.float32),
        jax.ShapeDtypeStruct((_L * _B, 128), jnp.float32),
    ),
    mesh=_mesh,
    scratch_types=[
        pltpu.VMEM((2, _UR), jnp.int32),
        pltpu.VMEM((2, _UR, _CTX), jnp.float32),
        pltpu.SemaphoreType.DMA,
        pltpu.SemaphoreType.DMA,
        pltpu.SemaphoreType.DMA,
        pltpu.SemaphoreType.DMA,
    ],
    compiler_params=pltpu.CompilerParams(
        use_tc_tiling_on_sc=False, needs_layout_passes=False),
)(_body)


def kernel(topics, structure_abstracts, embedding):
    absidx = structure_abstracts.T.reshape(-1).astype(jnp.int32)
    o1, o2 = _gather_tanh(topics.astype(jnp.int32), absidx, embedding)
    out1 = o1[:, :_CTX].reshape(_B, 1, _CTX)
    out2 = o2[:, :_CTX].reshape(_L, _B, _CTX).transpose(1, 0, 2)
    return (out1, out2)


# speedup vs baseline: 1.8298x; 1.0691x over previous
import functools

import jax
import jax.numpy as jnp
from jax import lax
from jax.experimental import pallas as pl
from jax.experimental.pallas import tpu as pltpu
from jax.experimental.pallas import tpu_sc as plsc

_B = 4096
_L = 200
_CTX = 32
_NW = 32
_UR = 1024
_NU2 = _L * _B // _UR
_NU = _NU2 + _B // _UR
_UPW = -(-_NU // _NW)


def _tanh16(x):
    e = jnp.exp(x + x)
    return 1.0 - 2.0 / (e + 1.0)


def _body(topics_hbm, absidx_hbm, table_hbm, out1_hbm, out2_hbm,
          idx_v, g_v, sem0, sem1, osem0, osem1):
    wid = lax.axis_index("s") * 2 + lax.axis_index("c")
    gsems = (sem0, sem1)
    osems = (osem0, osem1)

    def load_idx(k, buf):
        u = wid + _NW * k

        @pl.when(u < _NU2)
        def _():
            pltpu.sync_copy(absidx_hbm.at[pl.ds(u * _UR, _UR)],
                            idx_v.at[buf])

        @pl.when(jnp.logical_and(u >= _NU2, u < _NU))
        def _():
            pltpu.sync_copy(topics_hbm.at[pl.ds((u - _NU2) * _UR, _UR)],
                            idx_v.at[buf])

    def start_gather(k, buf):
        @pl.when(wid + _NW * k < _NU)
        def _():
            pltpu.async_copy(table_hbm.at[idx_v.at[buf]], g_v.at[buf],
                             gsems[buf])

    def wait_gather(buf):
        pltpu.make_async_copy(table_hbm.at[idx_v.at[buf]], g_v.at[buf],
                              gsems[buf]).wait()

    def write_out(k, buf):
        u = wid + _NW * k

        @pl.when(u < _NU2)
        def _():
            pltpu.async_copy(
                g_v.at[buf],
                out2_hbm.at[pl.ds(u * _UR, _UR), pl.ds(0, _CTX)],
                osems[buf])

        @pl.when(jnp.logical_and(u >= _NU2, u < _NU))
        def _():
            pltpu.async_copy(
                g_v.at[buf],
                out1_hbm.at[pl.ds((u - _NU2) * _UR, _UR), pl.ds(0, _CTX)],
                osems[buf])

    def wait_out(k, buf):
        u = wid + _NW * k

        @pl.when(u < _NU2)
        def _():
            pltpu.make_async_copy(
                g_v.at[buf],
                out2_hbm.at[pl.ds(u * _UR, _UR), pl.ds(0, _CTX)],
                osems[buf]).wait()

        @pl.when(jnp.logical_and(u >= _NU2, u < _NU))
        def _():
            pltpu.make_async_copy(
                g_v.at[buf],
                out1_hbm.at[pl.ds((u - _NU2) * _UR, _UR), pl.ds(0, _CTX)],
                osems[buf]).wait()

    load_idx(0, 0)
    start_gather(0, 0)

    def unit_pair_iter(k2, carry):
        for b in range(2):
            k = 2 * k2 + b
            nb = 1 - b
            u = wid + _NW * k

            @pl.when(k >= 1)
            def _(k=k, nb=nb):
                wait_out(k - 1, nb)

            @pl.when(u + _NW < _NU)
            def _(k=k, nb=nb):
                load_idx(k + 1, nb)

            start_gather(k + 1, nb)

            @pl.when(u < _NU)
            def _run_unit(k=k, b=b):
                wait_gather(b)

                def row_iter(j, c):
                    r0 = j * 8
                    for t in range(8):
                        for h in range(2):
                            sl = (b, r0 + t, pl.ds(16 * h, 16))
                            g_v[sl] = _tanh16(g_v[sl])
                    return c

                lax.fori_loop(0, _UR // 8, row_iter, 0)
                write_out(k, b)

        return carry

    lax.fori_loop(0, _UPW // 2, unit_pair_iter, 0)
    wait_out(_UPW - 1, (_UPW - 1) % 2)


_mesh = plsc.VectorSubcoreMesh(core_axis_name="c", subcore_axis_name="s")

_gather_tanh = functools.partial(
    pl.kernel,
    out_type=(
        jax.ShapeDtypeStruct((_B, 128), jnp.float32),
        jax.ShapeDtypeStruct((_L * _B, 128), jnp.float32),
    ),
    mesh=_mesh,
    scratch_types=[
        pltpu.VMEM((2, _UR), jnp.int32),
        pltpu.VMEM((2, _UR, _CTX), jnp.float32),
        pltpu.SemaphoreType.DMA,
        pltpu.SemaphoreType.DMA,
        pltpu.SemaphoreType.DMA,
        pltpu.SemaphoreType.DMA,
    ],
    compiler_params=pltpu.CompilerParams(
        use_tc_tiling_on_sc=False, needs_layout_passes=False),
)(_body)


def kernel(topics, structure_abstracts, embedding):
    absidx = structure_abstracts.T.reshape(-1).astype(jnp.int32)
    o1, o2 = _gather_tanh(topics.astype(jnp.int32), absidx, embedding)
    out1 = o1[:, :_CTX].reshape(_B, 1, _CTX)
    out2 = o2[:, :_CTX].reshape(_L, _B, _CTX).transpose(1, 0, 2)
    return (out1, out2)


# speedup vs baseline: 1.8539x; 1.0131x over previous
import functools

import jax
import jax.numpy as jnp
from jax import lax
from jax.experimental import pallas as pl
from jax.experimental.pallas import tpu as pltpu
from jax.experimental.pallas import tpu_sc as plsc

_B = 4096
_L = 200
_CTX = 32
_NW = 32
_UR = 1024
_APW = _L * _B // _UR // _NW
_UPW = _APW + 1
_IDXPW = _APW * _UR
_NTW = _B // _UR


def _tanh16(x):
    e = jnp.exp(x + x)
    return 1.0 - 2.0 / (e + 1.0)


def _body(topics_hbm, absidx_hbm, table_hbm, out1_hbm, out2_hbm,
          idx_v, g_v, sem0, sem1, osem0, osem1):
    wid = lax.axis_index("s") * 2 + lax.axis_index("c")
    gsems = (sem0, sem1)
    osems = (osem0, osem1)

    def unit_valid(k):
        return jnp.logical_or(k < _APW,
                              jnp.logical_and(k < _UPW, wid < _NTW))

    pltpu.sync_copy(absidx_hbm.at[pl.ds(wid * _IDXPW, _IDXPW)],
                    idx_v.at[pl.ds(0, _IDXPW)])

    @pl.when(wid < _NTW)
    def _():
        pltpu.sync_copy(topics_hbm.at[pl.ds(wid * _UR, _UR)],
                        idx_v.at[pl.ds(_IDXPW, _UR)])

    def start_gather(k, buf):
        @pl.when(unit_valid(k))
        def _():
            pltpu.async_copy(
                table_hbm.at[idx_v.at[pl.ds(k * _UR, _UR)]],
                g_v.at[buf], gsems[buf])

    def wait_gather(k, buf):
        pltpu.make_async_copy(
            table_hbm.at[idx_v.at[pl.ds(k * _UR, _UR)]],
            g_v.at[buf], gsems[buf]).wait()

    def write_out(k, buf):
        @pl.when(k < _APW)
        def _():
            pltpu.async_copy(
                g_v.at[buf],
                out2_hbm.at[pl.ds((wid * _APW + k) * _UR, _UR),
                            pl.ds(0, _CTX)],
                osems[buf])

        @pl.when(k >= _APW)
        def _():
            pltpu.async_copy(
                g_v.at[buf],
                out1_hbm.at[pl.ds(wid * _UR, _UR), pl.ds(0, _CTX)],
                osems[buf])

    def wait_out(k, buf):
        @pl.when(k < _APW)
        def _():
            pltpu.make_async_copy(
                g_v.at[buf],
                out2_hbm.at[pl.ds((wid * _APW + k) * _UR, _UR),
                            pl.ds(0, _CTX)],
                osems[buf]).wait()

        @pl.when(k >= _APW)
        def _():
            pltpu.make_async_copy(
                g_v.at[buf],
                out1_hbm.at[pl.ds(wid * _UR, _UR), pl.ds(0, _CTX)],
                osems[buf]).wait()

    start_gather(0, 0)

    def unit_pair_iter(k2, carry):
        for b in range(2):
            k = 2 * k2 + b
            nb = 1 - b

            @pl.when(k >= 1)
            def _(k=k, nb=nb):
                wait_out(k - 1, nb)

            start_gather(k + 1, nb)

            @pl.when(unit_valid(k))
            def _run_unit(k=k, b=b):
                wait_gather(k, b)

                def row_iter(j, c):
                    r0 = j * 16
                    for t in range(16):
                        for h in range(2):
                            sl = (b, r0 + t, pl.ds(16 * h, 16))
                            g_v[sl] = _tanh16(g_v[sl])
                    return c

                lax.fori_loop(0, _UR // 16, row_iter, 0)
                write_out(k, b)

        return carry

    lax.fori_loop(0, _UPW // 2, unit_pair_iter, 0)
    @pl.when(wid < _NTW)
    def _():
        wait_out(_UPW - 1, (_UPW - 1) % 2)


_mesh = plsc.VectorSubcoreMesh(core_axis_name="c", subcore_axis_name="s")

_gather_tanh = functools.partial(
    pl.kernel,
    out_type=(
        jax.ShapeDtypeStruct((_B, 128), jnp.float32),
        jax.ShapeDtypeStruct((_L * _B, 128), jnp.float32),
    ),
    mesh=_mesh,
    scratch_types=[
        pltpu.VMEM((_IDXPW + _UR,), jnp.int32),
        pltpu.VMEM((2, _UR, _CTX), jnp.float32),
        pltpu.SemaphoreType.DMA,
        pltpu.SemaphoreType.DMA,
        pltpu.SemaphoreType.DMA,
        pltpu.SemaphoreType.DMA,
    ],
    compiler_params=pltpu.CompilerParams(
        use_tc_tiling_on_sc=False, needs_layout_passes=False),
)(_body)


def kernel(topics, structure_abstracts, embedding):
    absidx = structure_abstracts.T.reshape(-1).astype(jnp.int32)
    o1, o2 = _gather_tanh(topics.astype(jnp.int32), absidx, embedding)
    out1 = o1[:, :_CTX].reshape(_B, 1, _CTX)
    out2 = o2[:, :_CTX].reshape(_L, _B, _CTX).transpose(1, 0, 2)
    return (out1, out2)
